# Initial kernel scaffold; baseline (speedup 1.0000x reference)
import jax
import jax.numpy as jnp
from jax.experimental import pallas as pl


def kernel(x, edge_index, W, b):
    raise NotImplementedError("write your pallas kernel here")


# speedup vs baseline: 4.5555x; 4.5555x over previous
import functools

import jax
import jax.numpy as jnp
from jax import lax
from jax.experimental import pallas as pl
from jax.experimental.pallas import tpu as pltpu
from jax.experimental.pallas import tpu_sc as plsc

NC = 2
NS = 16
NW = NC * NS
LANES = 16

CHUNK = 128
GRP = 8
NGRP = 10
ITERS = GRP * NGRP
NPAD = 10240


def _sc_aggregate(x, src4, dst4):
    n, d = x.shape
    rpt = NPAD // NS
    assert src4.shape == (NW, NGRP, GRP, CHUNK)

    mesh = plsc.VectorSubcoreMesh(
        core_axis_name="c", subcore_axis_name="s", num_cores=NC, num_subcores=NS
    )

    @functools.partial(
        pl.kernel,
        out_type=(
            jax.ShapeDtypeStruct((NC, NPAD, d), jnp.float32),
            jax.ShapeDtypeStruct((NC, NS, NPAD), jnp.float32),
        ),
        mesh=mesh,
        compiler_params=pltpu.CompilerParams(needs_layout_passes=False),
        scratch_types=(
            pltpu.VMEM((GRP, CHUNK), jnp.int32),
            pltpu.VMEM((GRP, CHUNK), jnp.int32),
            pltpu.VMEM((GRP, CHUNK), jnp.int32),
            pltpu.VMEM((GRP, CHUNK), jnp.int32),
            pltpu.VMEM((CHUNK, d), jnp.float32),
            pltpu.VMEM((CHUNK, d), jnp.float32),
            pltpu.VMEM((NPAD,), jnp.float32),
            pltpu.VMEM_SHARED((NPAD, d), jnp.float32),
            pltpu.SemaphoreType.DMA,
            pltpu.SemaphoreType.DMA,
        ),
    )
    def agg(x_hbm, src_hbm, dst_hbm, psum_hbm, pdeg_hbm,
            src_e, src_o, dst_e, dst_o, buf_a, buf_b, deg_v, acc,
            sem_a, sem_b):
        cid = lax.axis_index("c")
        sid = lax.axis_index("s")
        wid = sid * NC + cid
        zvec = jnp.zeros((LANES,), jnp.float32)
        ones = jnp.ones((LANES,), jnp.float32)

        def fill_deg(i, carry):
            deg_v[pl.ds(i * LANES, LANES)] = zvec
            return carry
        lax.fori_loop(0, NPAD // LANES, fill_deg, 0)

        def fill_rows(i, carry):
            for j in range(d // LANES):
                buf_a[i, pl.ds(j * LANES, LANES)] = zvec
            return carry
        lax.fori_loop(0, CHUNK, fill_rows, 0)

        for c in range(rpt // CHUNK):
            pltpu.sync_copy(buf_a, acc.at[pl.ds(sid * rpt + c * CHUNK, CHUNK)])
        plsc.subcore_barrier()

        bufs = (buf_a, buf_b)
        sems = (sem_a, sem_b)
        srcs = (src_e, src_o)
        dsts = (dst_e, dst_o)

        def gather(src_grp, c, buf, sem):
            pltpu.async_copy(x_hbm.at[src_grp.at[c]], buf, sem)

        def consume(dst_grp, c, buf, sem):
            pltpu.make_async_copy(x_hbm, buf, sem).wait()
            pltpu.sync_copy(buf, acc.at[dst_grp.at[c]], add=True)
            for k in range(CHUNK // LANES):
                dvec = dst_grp[c, pl.ds(k * LANES, LANES)]
                plsc.addupdate_scatter(deg_v, [dvec], ones)

        pltpu.sync_copy(src_hbm.at[wid, 0], src_e)
        pltpu.sync_copy(dst_hbm.at[wid, 0], dst_e)
        pltpu.sync_copy(src_hbm.at[wid, 1], src_o)
        pltpu.sync_copy(dst_hbm.at[wid, 1], dst_o)
        gather(src_e, 0, buf_a, sem_a)
        gather(src_e, 1, buf_b, sem_b)

        def step(g2, carry):
            for par in range(2):
                sg, dg = srcs[par], dsts[par]
                sg_n, dg_n = srcs[1 - par], dsts[1 - par]
                for c in range(GRP):
                    b = c % 2
                    consume(dg, c, bufs[b], sems[b])
                    if c < GRP - 2:
                        gather(sg, c + 2, bufs[b], sems[b])
                    elif par == 0:
                        gather(sg_n, c - (GRP - 2), bufs[b], sems[b])
                    else:
                        @pl.when(g2 + 1 < NGRP // 2)
                        def _pf():
                            gather(sg_n, c - (GRP - 2), bufs[b], sems[b])
                @pl.when(2 * g2 + par + 2 < NGRP)
                def _refill():
                    grp = 2 * g2 + par + 2
                    pltpu.sync_copy(src_hbm.at[wid, grp], sg)
                    pltpu.sync_copy(dst_hbm.at[wid, grp], dg)
            return carry
        lax.fori_loop(0, NGRP // 2, step, 0)

        plsc.subcore_barrier()

        pltpu.sync_copy(deg_v, pdeg_hbm.at[cid, sid])
        pltpu.sync_copy(acc.at[pl.ds(sid * rpt, rpt)],
                        psum_hbm.at[cid, pl.ds(sid * rpt, rpt)])

    return agg(x, src4, dst4)


def _tc_finish(psum, pdeg, W, b2d):
    _, npad, d = psum.shape
    dout = W.shape[0]
    rblk = 1024
    grid = (npad // rblk,)

    def body(ps_ref, pd_ref, w_ref, b_ref, o_ref):
        s = ps_ref[0] + ps_ref[1]
        deg = jnp.sum(pd_ref[...], axis=(0, 1)).reshape(rblk, 1)
        mean = s / (deg + 1e-6)
        o_ref[...] = lax.dot_general(
            mean, w_ref[...], (((1,), (1,)), ((), ())),
            preferred_element_type=jnp.float32) + b_ref[...]

    return pl.pallas_call(
        body,
        grid=grid,
        in_specs=[
            pl.BlockSpec((NC, rblk, d), lambda i: (0, i, 0)),
            pl.BlockSpec((NC, NS, rblk), lambda i: (0, 0, i)),
            pl.BlockSpec((dout, d), lambda i: (0, 0)),
            pl.BlockSpec((1, dout), lambda i: (0, 0)),
        ],
        out_specs=pl.BlockSpec((rblk, dout), lambda i: (i, 0)),
        out_shape=jax.ShapeDtypeStruct((npad, dout), jnp.float32),
    )(psum, pdeg, W, b2d)


@jax.jit
def kernel(x, edge_index, W, b):
    n = x.shape[0]
    e = edge_index.shape[1]
    epad = NW * ITERS * CHUNK
    dst = edge_index[0].astype(jnp.int32)
    src = edge_index[1].astype(jnp.int32)
    src4 = jnp.concatenate(
        [src, jnp.zeros((epad - e,), jnp.int32)]
    ).reshape(NW, NGRP, GRP, CHUNK)
    dst4 = jnp.concatenate(
        [dst, jnp.full((epad - e,), n, jnp.int32)]
    ).reshape(NW, NGRP, GRP, CHUNK)
    psum, pdeg = _sc_aggregate(x, src4, dst4)
    out = _tc_finish(psum, pdeg, W, b.reshape(1, -1))
    return out[:n]
